# speedup vs baseline: 1.9556x; 1.9556x over previous
import jax
import jax.numpy as jnp
from jax.experimental import pallas as pl
from jax.experimental.pallas import tpu as pltpu

B, S, D, F, T = 2, 4096, 1024, 4096, 4
TM = 2048
TF = 512


def _ffn_kernel(tid_ref, x_ref, win_ref, bin_ref, wout_ref, bout_ref,
                dwi_ref, dbi_ref, dwo_ref, dbo_ref, out_ref):
    f = pl.program_id(1)
    wi = win_ref[...] + dwi_ref[0]
    bi = bin_ref[...] + dbi_ref[0]
    h = jax.lax.dot_general(
        x_ref[...], wi, (((1,), (1,)), ((), ())),
        preferred_element_type=jnp.float32)
    h = jax.nn.gelu(h + bi)
    wo = wout_ref[...] + dwo_ref[0]
    acc = jax.lax.dot_general(
        h, wo, (((1,), (1,)), ((), ())),
        preferred_element_type=jnp.float32)

    @pl.when(f == 0)
    def _init():
        out_ref[...] = acc + (bout_ref[...] + dbo_ref[0])

    @pl.when(f != 0)
    def _acc():
        out_ref[...] += acc


def kernel(x, W_in, b_in, W_out, b_out, dW_in, db_in, dW_out, db_out, task_id):
    xm = x.reshape(B * S, D)
    b_in2 = b_in.reshape(1, F)
    b_out2 = b_out.reshape(1, D)
    db_in3 = db_in.reshape(T, 1, F)
    db_out3 = db_out.reshape(T, 1, D)
    tid = jnp.asarray(task_id, jnp.int32).reshape(1)

    nm = (B * S) // TM
    nf = F // TF
    grid_spec = pltpu.PrefetchScalarGridSpec(
        num_scalar_prefetch=1,
        grid=(nm, nf),
        in_specs=[
            pl.BlockSpec((TM, D), lambda m, f, t: (m, 0)),
            pl.BlockSpec((TF, D), lambda m, f, t: (f, 0)),
            pl.BlockSpec((1, TF), lambda m, f, t: (0, f)),
            pl.BlockSpec((D, TF), lambda m, f, t: (0, f)),
            pl.BlockSpec((1, D), lambda m, f, t: (0, 0)),
            pl.BlockSpec((1, TF, D), lambda m, f, t: (t[0], f, 0)),
            pl.BlockSpec((1, 1, TF), lambda m, f, t: (t[0], 0, f)),
            pl.BlockSpec((1, D, TF), lambda m, f, t: (t[0], 0, f)),
            pl.BlockSpec((1, 1, D), lambda m, f, t: (t[0], 0, 0)),
        ],
        out_specs=pl.BlockSpec((TM, D), lambda m, f, t: (m, 0)),
    )
    out = pl.pallas_call(
        _ffn_kernel,
        grid_spec=grid_spec,
        out_shape=jax.ShapeDtypeStruct((B * S, D), jnp.float32),
        compiler_params=pltpu.CompilerParams(
            dimension_semantics=("parallel", "arbitrary")),
    )(tid, xm, W_in, b_in2, W_out, b_out2, dW_in, db_in3, dW_out, db_out3)
    return out.reshape(B, S, D)
